# baseline (speedup 1.0000x reference)
import functools

import jax
import jax.numpy as jnp
from jax import lax
from jax.experimental import pallas as pl
from jax.experimental.pallas import tpu as pltpu
from jax.experimental.pallas import tpu_sc as plsc

BATCH = 16384
NUM_EMB = 1000000
HIST = 50
DIM = 64
LANES = 16
NUM_CORES = 2
NUM_SUBCORES = 16
NUM_WORKERS = NUM_CORES * NUM_SUBCORES
ROWS_PER_WORKER = BATCH // NUM_WORKERS
XCOLS = 128
GHIST = 56
CHUNK = 16
IDX_PER_CHUNK = CHUNK * GHIST
NUM_CHUNKS = ROWS_PER_WORKER // CHUNK
COMPACT_OFFS = (0, 16, 32, 40)


def _encoder_kernel(x_hbm, tab_hbm, out_hbm, xraw0, xraw1, idx0, idx1,
                    rows0, rows1, acc_v, sem0, sem1):
    wid = lax.axis_index("s") * NUM_CORES + lax.axis_index("c")
    base = wid * ROWS_PER_WORKER
    bufs = ((xraw0, idx0, rows0, sem0), (xraw1, idx1, rows1, sem1))

    def start_gather(ch, buf):
        xraw_v, idx_v, rows_v, sem = buf
        pltpu.sync_copy(x_hbm.at[pl.ds(base + ch * CHUNK, CHUNK)], xraw_v)
        for c in range(CHUNK):
            for off in COMPACT_OFFS:
                v = xraw_v[c, pl.ds(off, LANES)]
                idx_v[pl.ds(c * GHIST + off, LANES)] = v + v
        pltpu.async_copy(tab_hbm.at[idx_v], rows_v, sem)

    start_gather(0, bufs[0])

    @pl.loop(0, NUM_CHUNKS, step=2)
    def _(ch):
        for b in range(2):
            cur = ch + b
            _, idx_v, rows_v, sem = bufs[b]

            @pl.when(cur + 1 < NUM_CHUNKS)
            def _():
                start_gather(cur + 1, bufs[b ^ 1])

            pltpu.make_async_copy(tab_hbm.at[idx_v], rows_v, sem).wait()

            @pl.loop(0, CHUNK)
            def _(c):
                slices = [pl.ds(d * 2 * LANES, 2 * LANES)
                          for d in range(DIM // (2 * LANES))]
                for sl in slices:
                    vals = [rows_v[c * GHIST + 2 * l, sl]
                            + rows_v[c * GHIST + 2 * l + 1, sl]
                            for l in range(HIST // 2)]
                    while len(vals) > 1:
                        vals = [vals[i] + vals[i + 1]
                                if i + 1 < len(vals) else vals[i]
                                for i in range(0, len(vals), 2)]
                    acc_v[c, sl] = vals[0]

            pltpu.sync_copy(acc_v, out_hbm.at[pl.ds(base + cur * CHUNK, CHUNK)])


def kernel(x, table):
    mesh = plsc.VectorSubcoreMesh(core_axis_name="c", subcore_axis_name="s")
    run = functools.partial(
        pl.kernel,
        out_type=jax.ShapeDtypeStruct((BATCH, DIM), jnp.bfloat16),
        mesh=mesh,
        scratch_types=[
            pltpu.VMEM((CHUNK, XCOLS), jnp.int32),
            pltpu.VMEM((CHUNK, XCOLS), jnp.int32),
            pltpu.VMEM((IDX_PER_CHUNK,), jnp.int32),
            pltpu.VMEM((IDX_PER_CHUNK,), jnp.int32),
            pltpu.VMEM((IDX_PER_CHUNK, DIM), jnp.bfloat16),
            pltpu.VMEM((IDX_PER_CHUNK, DIM), jnp.bfloat16),
            pltpu.VMEM((CHUNK, DIM), jnp.bfloat16),
            pltpu.SemaphoreType.DMA,
            pltpu.SemaphoreType.DMA,
        ],
        compiler_params=pltpu.CompilerParams(use_tc_tiling_on_sc=False),
    )(_encoder_kernel)
    xf = jnp.pad(x.astype(jnp.int32), ((0, 0), (0, XCOLS - HIST)),
                 mode="edge")
    tp = jnp.pad(table.astype(jnp.bfloat16), ((0, 0), (0, XCOLS - DIM)))
    tp = tp.reshape(2 * NUM_EMB, DIM)
    return run(xf, tp).astype(jnp.float32)


# speedup vs baseline: 2.0346x; 2.0346x over previous
import functools

import jax
import jax.numpy as jnp
from jax import lax
from jax.experimental import pallas as pl
from jax.experimental.pallas import tpu as pltpu
from jax.experimental.pallas import tpu_sc as plsc

BATCH = 16384
NUM_EMB = 1000000
HIST = 50
DIM = 64
LANES = 16
NUM_CORES = 2
NUM_SUBCORES = 16
NUM_WORKERS = NUM_CORES * NUM_SUBCORES
ROWS_PER_WORKER = BATCH // NUM_WORKERS
XCOLS = 128
GHIST = 56
CHUNK = 16
IDX_PER_CHUNK = CHUNK * GHIST
NUM_CHUNKS = ROWS_PER_WORKER // CHUNK
COMPACT_OFFS = (0, 16, 32, 40)


def _encoder_kernel(x_hbm, tab_hbm, out_hbm, xraw0, xraw1, idx0, idx1,
                    rows0, rows1, acc_v, sem0, sem1):
    wid = lax.axis_index("s") * NUM_CORES + lax.axis_index("c")
    base = wid * ROWS_PER_WORKER
    bufs = ((xraw0, idx0, rows0, sem0), (xraw1, idx1, rows1, sem1))

    def start_gather(ch, buf):
        xraw_v, idx_v, rows_v, sem = buf
        pltpu.sync_copy(x_hbm.at[pl.ds(base + ch * CHUNK, CHUNK)], xraw_v)
        for c in range(CHUNK):
            for off in COMPACT_OFFS:
                v = xraw_v[c, pl.ds(off, LANES)]
                idx_v[pl.ds(c * GHIST + off, LANES)] = v + v
        pltpu.async_copy(tab_hbm.at[idx_v], rows_v, sem)

    start_gather(0, bufs[0])

    @pl.loop(0, NUM_CHUNKS, step=2)
    def _(ch):
        for b in range(2):
            cur = ch + b
            _, idx_v, rows_v, sem = bufs[b]

            @pl.when(cur + 1 < NUM_CHUNKS)
            def _():
                start_gather(cur + 1, bufs[b ^ 1])

            pltpu.make_async_copy(tab_hbm.at[idx_v], rows_v, sem).wait()

            @pl.loop(0, CHUNK)
            def _(c):
                slices = [pl.ds(d * LANES, LANES) for d in range(DIM // LANES)]
                accs = [rows_v[c * GHIST, sl] for sl in slices]
                for l in range(1, HIST):
                    for d, sl in enumerate(slices):
                        accs[d] = accs[d] + rows_v[c * GHIST + l, sl]
                for d, sl in enumerate(slices):
                    acc_v[c, sl] = accs[d]

            pltpu.sync_copy(acc_v, out_hbm.at[pl.ds(base + cur * CHUNK, CHUNK)])


def kernel(x, table):
    mesh = plsc.VectorSubcoreMesh(core_axis_name="c", subcore_axis_name="s")
    run = functools.partial(
        pl.kernel,
        out_type=jax.ShapeDtypeStruct((BATCH, DIM), jnp.float32),
        mesh=mesh,
        scratch_types=[
            pltpu.VMEM((CHUNK, XCOLS), jnp.int32),
            pltpu.VMEM((CHUNK, XCOLS), jnp.int32),
            pltpu.VMEM((IDX_PER_CHUNK,), jnp.int32),
            pltpu.VMEM((IDX_PER_CHUNK,), jnp.int32),
            pltpu.VMEM((IDX_PER_CHUNK, DIM), jnp.float32),
            pltpu.VMEM((IDX_PER_CHUNK, DIM), jnp.float32),
            pltpu.VMEM((CHUNK, DIM), jnp.float32),
            pltpu.SemaphoreType.DMA,
            pltpu.SemaphoreType.DMA,
        ],
        compiler_params=pltpu.CompilerParams(use_tc_tiling_on_sc=False),
    )(_encoder_kernel)
    xf = jnp.pad(x.astype(jnp.int32), ((0, 0), (0, XCOLS - HIST)),
                 mode="edge")
    tp = jnp.pad(table, ((0, 0), (0, XCOLS - DIM)))
    tp = tp.reshape(2 * NUM_EMB, DIM)
    return run(xf, tp)


# speedup vs baseline: 2.0374x; 1.0014x over previous
import functools

import jax
import jax.numpy as jnp
from jax import lax
from jax.experimental import pallas as pl
from jax.experimental.pallas import tpu as pltpu
from jax.experimental.pallas import tpu_sc as plsc

BATCH = 16384
NUM_EMB = 1000000
HIST = 50
DIM = 64
LANES = 16
NUM_CORES = 2
NUM_SUBCORES = 16
NUM_WORKERS = NUM_CORES * NUM_SUBCORES
ROWS_PER_WORKER = BATCH // NUM_WORKERS
XCOLS = 128
GHIST = 56
CHUNK = 16
IDX_PER_CHUNK = CHUNK * GHIST
NUM_CHUNKS = ROWS_PER_WORKER // CHUNK
COMPACT_OFFS = (0, 16, 32, 40)


def _encoder_kernel(x_hbm, tab_hbm, out_hbm, xraw0, xraw1, idx0, idx1,
                    rows0, rows1, acc0, acc1, sem0, sem1, semo0, semo1):
    wid = lax.axis_index("s") * NUM_CORES + lax.axis_index("c")
    base = wid * ROWS_PER_WORKER
    bufs = ((xraw0, idx0, rows0, sem0), (xraw1, idx1, rows1, sem1))
    accs_b = (acc0, acc1)
    semo_b = (semo0, semo1)

    def out_copy(ch, b):
        return pltpu.make_async_copy(
            accs_b[b], out_hbm.at[pl.ds(base + ch * CHUNK, CHUNK)], semo_b[b])

    def start_gather(ch, buf):
        xraw_v, idx_v, rows_v, sem = buf
        pltpu.sync_copy(x_hbm.at[pl.ds(base + ch * CHUNK, CHUNK)], xraw_v)
        for c in range(CHUNK):
            for off in COMPACT_OFFS:
                v = xraw_v[c, pl.ds(off, LANES)]
                idx_v[pl.ds(c * GHIST + off, LANES)] = v + v
        pltpu.async_copy(tab_hbm.at[idx_v], rows_v, sem)

    start_gather(0, bufs[0])

    @pl.loop(0, NUM_CHUNKS, step=2)
    def _(ch):
        for b in range(2):
            cur = ch + b
            _, idx_v, rows_v, sem = bufs[b]

            @pl.when(cur + 1 < NUM_CHUNKS)
            def _():
                start_gather(cur + 1, bufs[b ^ 1])

            pltpu.make_async_copy(tab_hbm.at[idx_v], rows_v, sem).wait()

            @pl.when(cur >= 2)
            def _():
                out_copy(cur - 2, b).wait()

            acc_v = accs_b[b]

            @pl.loop(0, CHUNK)
            def _(c):
                slices = [pl.ds(d * LANES, LANES) for d in range(DIM // LANES)]
                accs = [rows_v[c * GHIST, sl] for sl in slices]
                for l in range(1, HIST):
                    for d, sl in enumerate(slices):
                        accs[d] = accs[d] + rows_v[c * GHIST + l, sl]
                for d, sl in enumerate(slices):
                    acc_v[c, sl] = accs[d]

            out_copy(cur, b).start()

    out_copy(NUM_CHUNKS - 2, 0).wait()
    out_copy(NUM_CHUNKS - 1, 1).wait()


def kernel(x, table):
    mesh = plsc.VectorSubcoreMesh(core_axis_name="c", subcore_axis_name="s")
    run = functools.partial(
        pl.kernel,
        out_type=jax.ShapeDtypeStruct((BATCH, DIM), jnp.float32),
        mesh=mesh,
        scratch_types=[
            pltpu.VMEM((CHUNK, XCOLS), jnp.int32),
            pltpu.VMEM((CHUNK, XCOLS), jnp.int32),
            pltpu.VMEM((IDX_PER_CHUNK,), jnp.int32),
            pltpu.VMEM((IDX_PER_CHUNK,), jnp.int32),
            pltpu.VMEM((IDX_PER_CHUNK, DIM), jnp.float32),
            pltpu.VMEM((IDX_PER_CHUNK, DIM), jnp.float32),
            pltpu.VMEM((CHUNK, DIM), jnp.float32),
            pltpu.VMEM((CHUNK, DIM), jnp.float32),
            pltpu.SemaphoreType.DMA,
            pltpu.SemaphoreType.DMA,
            pltpu.SemaphoreType.DMA,
            pltpu.SemaphoreType.DMA,
        ],
        compiler_params=pltpu.CompilerParams(use_tc_tiling_on_sc=False),
    )(_encoder_kernel)
    xf = jnp.pad(x.astype(jnp.int32), ((0, 0), (0, XCOLS - HIST)),
                 mode="edge")
    tp = jnp.pad(table, ((0, 0), (0, XCOLS - DIM)))
    tp = tp.reshape(2 * NUM_EMB, DIM)
    return run(xf, tp)
